# Initial kernel scaffold; baseline (speedup 1.0000x reference)
import jax
import jax.numpy as jnp
from jax.experimental import pallas as pl


def kernel(verts, faces):
    raise NotImplementedError("write your pallas kernel here")


# speedup vs baseline: 37.2184x; 37.2184x over previous
import functools

import jax
import jax.numpy as jnp
from jax import lax
from jax.experimental import pallas as pl
from jax.experimental.pallas import tpu as pltpu
from jax.experimental.pallas import tpu_sc as plsc

NUM_SC = 2
NUM_TILES = 16
LANES = 16
RW = 8

W = 128
NSUB = 8


def _sc_laplacian(B, N, F, NPAD, FPAD):
    CH = NPAD // NUM_TILES
    FCH = CH // NSUB
    PER_TILE = FPAD // NUM_TILES
    NWIN = PER_TILE // W
    assert CH % NSUB == 0 and FCH % 2 == 0
    assert FPAD % (NUM_TILES * W) == 0 and W % 8 == 0
    assert B == NUM_SC * 2

    mesh = plsc.VectorSubcoreMesh(
        core_axis_name="c", subcore_axis_name="s",
        num_cores=NUM_SC, num_subcores=NUM_TILES,
    )

    @functools.partial(
        pl.kernel,
        mesh=mesh,
        compiler_params=pltpu.CompilerParams(
            needs_layout_passes=False, use_tc_tiling_on_sc=False),
        out_type=jax.ShapeDtypeStruct((B, NPAD, RW), jnp.float32),
        scratch_types=[
            pltpu.VMEM_SHARED((2 * NPAD, RW), jnp.float32),
            pltpu.VMEM((W,), jnp.int32),
            pltpu.VMEM((W,), jnp.int32),
            pltpu.VMEM((W,), jnp.int32),
            pltpu.VMEM((W,), jnp.int32),
            pltpu.VMEM((W,), jnp.int32),
            pltpu.VMEM((W,), jnp.int32),
            pltpu.VMEM((W, RW), jnp.float32),
            pltpu.VMEM((W, RW), jnp.float32),
            pltpu.VMEM((W, RW), jnp.float32),
            pltpu.VMEM((W, RW), jnp.float32),
            pltpu.VMEM((FCH, RW), jnp.float32),
            pltpu.VMEM((FCH, RW), jnp.float32),
            pltpu.VMEM((FCH, RW), jnp.float32),
        ],
    )
    def lap(v4_hbm, ft_hbm, fto_hbm, z_hbm, out_hbm, sbuf,
            idx0, idx1, idx2, ox0, ox1, ox2, r0, r1, r2, fsb,
            vchunk, schunk, ochunk):
        idxs = (idx0, idx1, idx2)
        oxs = (ox0, ox1, ox2)
        c = lax.axis_index("c")
        s = lax.axis_index("s")
        iota = lax.iota(jnp.int32, LANES)
        row2 = lax.shift_right_logical(iota, 3)
        col8 = lax.bitwise_and(iota, 7)
        lane3 = lax.bitwise_or(lax.bitwise_and(iota, 0), 3)

        for bl in range(2):
            b = c * 2 + bl

            pltpu.sync_copy(z_hbm, ochunk)

            @pl.loop(0, NSUB)
            def _stage(k):
                sub = pl.ds(s * CH + k * FCH, FCH)
                ssub = pl.ds(NPAD + s * CH + k * FCH, FCH)
                pltpu.sync_copy(ochunk, sbuf.at[ssub])
                pltpu.sync_copy(v4_hbm.at[b, sub], vchunk)
                pltpu.sync_copy(vchunk, sbuf.at[sub])

            plsc.subcore_barrier()

            base0 = s * PER_TILE

            @pl.loop(0, NWIN)
            def _win(w):
                base = base0 + w * W
                fbase = (b * 3) * FPAD + base
                for j in range(3):
                    pltpu.sync_copy(ft_hbm.at[pl.ds(fbase + j * FPAD, W)],
                                    idxs[j])
                    pltpu.sync_copy(fto_hbm.at[pl.ds(fbase + j * FPAD, W)],
                                    oxs[j])
                pltpu.sync_copy(sbuf.at[idx0], r0)
                pltpu.sync_copy(sbuf.at[idx1], r1)
                pltpu.sync_copy(sbuf.at[idx2], r2)

                @pl.loop(0, W * RW // LANES, unroll=8)
                def _fs(i):
                    row = i * 2 + row2
                    x0 = plsc.load_gather(r0, [row, col8])
                    x1 = plsc.load_gather(r1, [row, col8])
                    x2 = plsc.load_gather(r2, [row, col8])
                    plsc.store_scatter(fsb, [row, col8], x0 + x1 + x2)

                for j in range(3):
                    pltpu.sync_copy(fsb, sbuf.at[oxs[j]], add=True)

            plsc.subcore_barrier()

            @pl.loop(0, NSUB)
            def _fino(k):
                sub = pl.ds(s * CH + k * FCH, FCH)
                ssub = pl.ds(NPAD + s * CH + k * FCH, FCH)
                pltpu.sync_copy(sbuf.at[ssub], schunk)
                pltpu.sync_copy(sbuf.at[sub], vchunk)

                @pl.loop(0, FCH * RW // LANES, unroll=8)
                def _fin(i):
                    row = i * 2 + row2
                    sv = plsc.load_gather(schunk, [row, col8])
                    s3 = plsc.load_gather(schunk, [row, lane3])
                    vv = plsc.load_gather(vchunk, [row, col8])
                    o = (s3 * vv - sv) / (s3 * jnp.float32(2.0 / 3.0)
                                          + jnp.float32(1e-12))
                    plsc.store_scatter(ochunk, [row, col8], o)

                pltpu.sync_copy(ochunk, out_hbm.at[b, sub])

            if bl == 0:
                plsc.subcore_barrier()

    return lap


def kernel(verts, faces):
    B, N, D = verts.shape
    F = faces.shape[1]
    assert D == 3 and faces.shape == (B, F, 3)

    quantum = NUM_TILES * NSUB * 8
    NPAD = -(-(N + 64) // quantum) * quantum
    n_extra = NPAD - N
    FPAD = -(-F // (NUM_TILES * W)) * (NUM_TILES * W)

    v4 = jnp.concatenate(
        [verts, jnp.ones((B, N, 1), verts.dtype)], axis=-1)
    v4 = jnp.pad(v4, ((0, 0), (0, n_extra), (0, RW - 4)))

    spread = max(1, min(64, n_extra))
    pad_idx = N + (jnp.arange(FPAD - F, dtype=jnp.int32) % spread)
    ft = jnp.transpose(faces.astype(jnp.int32), (0, 2, 1))
    ft = jnp.concatenate(
        [ft, jnp.broadcast_to(pad_idx, (B, 3, FPAD - F))], axis=-1)
    ft = ft.reshape(-1)
    fto = ft + jnp.int32(NPAD)

    zchunk = jnp.zeros((NPAD // (NUM_TILES * NSUB), RW), jnp.float32)

    out8 = _sc_laplacian(B, N, F, NPAD, FPAD)(v4, ft, fto, zchunk)
    return out8[:, :N, :3]
